# baseline (speedup 1.0000x reference)
import functools
import numpy as np
import jax
import jax.numpy as jnp
from jax import lax
from jax.experimental import pallas as pl
from jax.experimental.pallas import tpu as pltpu
from jax.experimental.pallas import tpu_sc as plsc

_B, _T, _P, _D = 8, 16, 576, 768
_N = _B * _T

_WORDS = [0x8D744451, 0xB39A25C9, 0x587166EB, 0x27893CC9]

_CHUNK = 64
_NCH = _P // _CHUNK
_TOKROWS = 32
_NTW = _P // _TOKROWS

_NC, _NS = 2, 16
_NW = _NC * _NS
_FPW = _N // _NW


def _mask_bit(f):
    w = [np.uint32(v) for v in _WORDS]
    word = jnp.where(f < 32, w[0], jnp.where(f < 64, w[1], jnp.where(f < 96, w[2], w[3])))
    return (word >> (f.astype(jnp.uint32) % 32)) & 1


def _sc_body(x_hbm, tok_hbm, out_hbm, tokbuf, buf0, buf1, sem_t, sem_g, sem_s):
    wid = lax.axis_index("s") * _NC + lax.axis_index("c")

    tc = [
        pltpu.make_async_copy(tok_hbm, tokbuf.at[pl.ds(r, 1)], sem_t)
        for r in range(_TOKROWS)
    ]
    for c in tc:
        c.start()
    for c in tc:
        c.wait()

    for k in range(_FPW):
        f = wid * _FPW + k
        bit = _mask_bit(f)

        @pl.when(bit != 0)
        def _():
            cs = [
                pltpu.make_async_copy(
                    tokbuf, out_hbm.at[f, pl.ds(j * _TOKROWS, _TOKROWS)], sem_s
                )
                for j in range(_NTW)
            ]
            for c in cs:
                c.start()
            for c in cs:
                c.wait()

        @pl.when(bit == 0)
        def _():
            bufs = [buf0, buf1]
            g = [
                pltpu.make_async_copy(
                    x_hbm.at[f, pl.ds(j * _CHUNK, _CHUNK)], bufs[j % 2], sem_g
                )
                for j in range(_NCH)
            ]
            s = [
                pltpu.make_async_copy(
                    bufs[j % 2], out_hbm.at[f, pl.ds(j * _CHUNK, _CHUNK)], sem_s
                )
                for j in range(_NCH)
            ]
            g[0].start()
            for j in range(_NCH):
                if j + 1 < _NCH:
                    if j - 1 >= 0:
                        s[j - 1].wait()
                    g[j + 1].start()
                g[j].wait()
                s[j].start()
            if _NCH >= 2:
                s[_NCH - 2].wait()
            s[_NCH - 1].wait()


def kernel(x, mask_token):
    x3 = x.reshape(_N, _P, _D)
    tok = mask_token.reshape(1, _D)
    mesh = plsc.VectorSubcoreMesh(core_axis_name="c", subcore_axis_name="s")
    k = functools.partial(
        pl.kernel,
        mesh=mesh,
        out_type=jax.ShapeDtypeStruct((_N, _P, _D), jnp.float32),
        scratch_types=[
            pltpu.VMEM((_TOKROWS, _D), jnp.float32),
            pltpu.VMEM((_CHUNK, _D), jnp.float32),
            pltpu.VMEM((_CHUNK, _D), jnp.float32),
            pltpu.SemaphoreType.DMA,
            pltpu.SemaphoreType.DMA,
            pltpu.SemaphoreType.DMA,
        ],
    )(_sc_body)
    out3 = k(x3, tok)
    return out3.reshape(_B, _T, _P, _D)


# speedup vs baseline: 1.7067x; 1.7067x over previous
import numpy as np
import jax
import jax.numpy as jnp
from jax.experimental import pallas as pl
from jax.experimental.pallas import tpu as pltpu

_B, _T, _P, _D = 8, 16, 576, 768
_N = _B * _T

_WORDS = [0x8D744451, 0xB39A25C9, 0x587166EB, 0x27893CC9]
_FLAT = np.array([(w >> b) & 1 for w in _WORDS for b in range(32)], dtype=bool)
_MASKED = np.nonzero(_FLAT)[0]
_UNMASKED = np.nonzero(~_FLAT)[0]
_NCP = len(_UNMASKED)
_NTOK = len(_MASKED)

_K = 8
_A = 4


def _body(x_ref, tok_ref, out_ref, tokf, ring, sem_g, sem_s, sem_t):
    tokf[...] = jnp.broadcast_to(tok_ref[0, :], (_P, _D))

    g = [
        pltpu.make_async_copy(x_ref.at[int(f)], ring.at[c % _K], sem_g)
        for c, f in enumerate(_UNMASKED)
    ]
    s = [
        pltpu.make_async_copy(ring.at[c % _K], out_ref.at[int(f)], sem_s)
        for c, f in enumerate(_UNMASKED)
    ]
    t = [pltpu.make_async_copy(tokf, out_ref.at[int(f)], sem_t) for f in _MASKED]

    for c in range(_A):
        g[c].start()
    waited_s = -1
    for c in range(_NCP):
        g[c].wait()
        s[c].start()
        if c < _NTOK:
            t[c].start()
        if c + _A < _NCP:
            if c + _A - _K >= 0:
                s[c + _A - _K].wait()
                waited_s = c + _A - _K
            g[c + _A].start()
    for c in range(waited_s + 1, _NCP):
        s[c].wait()
    for c in range(_NCP, _NTOK):
        t[c].start()
    for c in range(_NTOK):
        t[c].wait()


def kernel(x, mask_token):
    x3 = x.reshape(_N, _P, _D)
    tok = mask_token.reshape(1, _D)
    out3 = pl.pallas_call(
        _body,
        in_specs=[
            pl.BlockSpec(memory_space=pl.ANY),
            pl.BlockSpec(memory_space=pltpu.VMEM),
        ],
        out_specs=pl.BlockSpec(memory_space=pl.ANY),
        out_shape=jax.ShapeDtypeStruct((_N, _P, _D), x.dtype),
        scratch_shapes=[
            pltpu.VMEM((_P, _D), jnp.float32),
            pltpu.VMEM((_K, _P, _D), jnp.float32),
            pltpu.SemaphoreType.DMA,
            pltpu.SemaphoreType.DMA,
            pltpu.SemaphoreType.DMA,
        ],
    )(x3, tok)
    return out3.reshape(_B, _T, _P, _D)
